# baseline (speedup 1.0000x reference)
import functools
import jax, jax.numpy as jnp
import numpy as np
from jax import lax
from jax.experimental import pallas as pl
from jax.experimental.pallas import tpu as pltpu
from jax.experimental.pallas import tpu_sc as plsc

_GRID = 256.0
_VOXELS = [4.0, 8.0, 16.0, 32.0]
_B = 16
_N = 10000
_NP = 10240
_E = 160000
_NC = 2
_NS = 16
_NW = _NC * _NS
_NT = _NP // _NW
_S = 2048
_K = 16


def _mm_body(x_ref, w_ref, o_ref):
    o_ref[...] = jnp.dot(x_ref[...], w_ref[...], preferred_element_type=jnp.float32)


def _pmatmul(x, w):
    M, Kd = x.shape
    co = w.shape[1]
    return pl.pallas_call(
        _mm_body,
        grid=(M // 1024,),
        in_specs=[pl.BlockSpec((1024, Kd), lambda i: (i, 0)),
                  pl.BlockSpec((Kd, co), lambda i: (0, 0))],
        out_specs=pl.BlockSpec((1024, co), lambda i: (i, 0)),
        out_shape=jax.ShapeDtypeStruct((M, co), jnp.float32),
    )(x, w)


_EP = 163840
_EW = _EP // _NW
_CH = 4
_EH = _EW // _CH
_KG = 32
_NG = _EH // _KG
_ZR = 8


def _agg_body(F, NCH, refs):
    xw_refs = refs[:NCH]
    (srcr_ref, dstr_ref, ea0_ref, ea1_ref, ea2_ref, we_ref, out_ref,
     sidx, didx, e0b, e1b, e2b, rows_a, rows_b, zbuf, wev, acc,
     sem_a, sem_b) = refs[NCH:]
    cid = lax.axis_index("c")
    sid = lax.axis_index("s")
    wid = sid * _NC + cid
    G = F // 16
    zero16 = jnp.zeros((16,), jnp.float32)
    pib = 'promise_in_bounds'
    ZS = _NP // _NS

    def zr(r, c):
        for g in range(G):
            zbuf[r, pl.ds(g * 16, 16)] = zero16
        return c
    lax.fori_loop(0, _ZR, zr, 0)

    for k in range(NCH):
        xw_ref = xw_refs[k]
        pltpu.sync_copy(we_ref.at[pl.ds(0, 3), pl.ds(k * F, F)], wev)
        def zc(t, c):
            pltpu.sync_copy(zbuf, acc.at[pl.ds(sid * ZS + t * _ZR, _ZR)])
            return c
        lax.fori_loop(0, ZS // _ZR, zc, 0)
        plsc.subcore_barrier()

        def chunk(h, c0_):
            pltpu.sync_copy(srcr_ref.at[wid, pl.ds(h * _NG, _NG)], sidx)
            pltpu.sync_copy(dstr_ref.at[wid, pl.ds(h * _NG, _NG)], didx)
            pltpu.sync_copy(ea0_ref.at[wid, pl.ds(h * _EH, _EH)], e0b)
            pltpu.sync_copy(ea1_ref.at[wid, pl.ds(h * _EH, _EH)], e1b)
            pltpu.sync_copy(ea2_ref.at[wid, pl.ds(h * _EH, _EH)], e2b)
            def proc(j, rows):
                wevs = [[wev[kk, pl.ds(g * 16, 16)] for g in range(G)] for kk in range(3)]
                for sub in range(_KG // 16):
                    e0v = e0b[pl.ds(j * _KG + sub * 16, 16)]
                    e1v = e1b[pl.ds(j * _KG + sub * 16, 16)]
                    e2v = e2b[pl.ds(j * _KG + sub * 16, 16)]

                    def edge(e16, c2):
                        lane = jnp.full((16,), e16, jnp.int32)
                        c0 = e0v.at[lane].get(mode=pib)
                        c1 = e1v.at[lane].get(mode=pib)
                        c2v = e2v.at[lane].get(mode=pib)
                        r = sub * 16 + e16
                        for g in range(G):
                            coef = c0 * wevs[0][g] + c1 * wevs[1][g] + c2v * wevs[2][g]
                            rows[r, pl.ds(g * 16, 16)] = rows[r, pl.ds(g * 16, 16)] * coef
                        return c2
                    lax.fori_loop(0, 16, edge, 0)
                pltpu.sync_copy(rows, acc.at[didx.at[j]], add=True)

            def grp1(j, c):
                pltpu.async_copy(xw_ref.at[sidx.at[j]], rows_a, sem_a).wait()
                proc(j, rows_a)
                return c
            lax.fori_loop(0, _NG, grp1, 0)
            return c0_
        lax.fori_loop(0, _CH, chunk, 0)
        plsc.subcore_barrier()
        pltpu.sync_copy(acc.at[pl.ds(sid * ZS, ZS)], out_ref.at[k, cid, pl.ds(sid * ZS, ZS)])


def _sc_agg(xw_chunks, srcr, dstr, ea0r, ea1r, ea2r, we):
    F = xw_chunks[0].shape[1]
    NCH = len(xw_chunks)
    mesh = plsc.VectorSubcoreMesh(core_axis_name="c", subcore_axis_name="s")
    k = functools.partial(
        pl.kernel,
        mesh=mesh,
        out_type=jax.ShapeDtypeStruct((NCH, _NC, _NP, F), jnp.float32),
        scratch_types=[
            pltpu.VMEM((_NG, _KG), jnp.int32),
            pltpu.VMEM((_NG, _KG), jnp.int32),
            pltpu.VMEM((_EH,), jnp.float32),
            pltpu.VMEM((_EH,), jnp.float32),
            pltpu.VMEM((_EH,), jnp.float32),
            pltpu.VMEM((_KG, F), jnp.float32),
            pltpu.VMEM((_KG, F), jnp.float32),
            pltpu.VMEM((_ZR, F), jnp.float32),
            pltpu.VMEM((3, F), jnp.float32),
            pltpu.VMEM_SHARED((_NP, F), jnp.float32),
            pltpu.SemaphoreType.DMA,
            pltpu.SemaphoreType.DMA,
        ],
    )(lambda *refs: _agg_body(F, NCH, refs))
    return k(*xw_chunks, srcr, dstr, ea0r, ea1r, ea2r, we)


_GC = 128
_NGC = _EW // _GC


def _prep_body(inv_ref, px_ref, py_ref, pz_ref, srcf_ref, dstf_ref,
               so_ref, do_ref, r0_ref, r1_ref, r2_ref, pm_ref,
               sfl, dfl, snew, dnew, ps, pd, mxb, sem):
    cid = lax.axis_index("c")
    sid = lax.axis_index("s")
    wid = sid * _NC + cid

    pltpu.sync_copy(srcf_ref.at[wid], sfl)
    pltpu.sync_copy(dstf_ref.at[wid], dfl)

    def gat(table, idx, out):
        cps = []
        for t in range(_NGC):
            cps.append(pltpu.async_copy(table.at[idx.at[pl.ds(t * _GC, _GC)]],
                                        out.at[pl.ds(t * _GC, _GC)], sem))
        for c in cps:
            c.wait()

    gat(inv_ref, sfl, snew)
    gat(inv_ref, dfl, dnew)
    pltpu.sync_copy(snew, so_ref.at[wid])
    pltpu.sync_copy(dnew, do_ref.at[wid])

    maxv = jnp.zeros((16,), jnp.float32)
    for comp, (pref, rref) in enumerate([(px_ref, r0_ref), (py_ref, r1_ref), (pz_ref, r2_ref)]):
        gat(pref, snew, ps)
        gat(pref, dnew, pd)
        def rloop(t, mv):
            a = pd[pl.ds(t * 16, 16)] - ps[pl.ds(t * 16, 16)]
            ps[pl.ds(t * 16, 16)] = a
            return jnp.maximum(mv, jnp.abs(a))
        maxv = lax.fori_loop(0, _EW // 16, rloop, maxv)
        pltpu.sync_copy(ps, rref.at[wid])
    mxb[pl.ds(0, 16)] = maxv
    pltpu.sync_copy(mxb, pm_ref.at[wid])


def _sc_prep(inv, px, py, pz, srcf, dstf):
    mesh = plsc.VectorSubcoreMesh(core_axis_name="c", subcore_axis_name="s")
    k = functools.partial(
        pl.kernel,
        mesh=mesh,
        out_type=(jax.ShapeDtypeStruct((_NW, _EW), jnp.int32),
                  jax.ShapeDtypeStruct((_NW, _EW), jnp.int32),
                  jax.ShapeDtypeStruct((_NW, _EW), jnp.float32),
                  jax.ShapeDtypeStruct((_NW, _EW), jnp.float32),
                  jax.ShapeDtypeStruct((_NW, _EW), jnp.float32),
                  jax.ShapeDtypeStruct((_NW, 16), jnp.float32)),
        scratch_types=[
            pltpu.VMEM((_EW,), jnp.int32),
            pltpu.VMEM((_EW,), jnp.int32),
            pltpu.VMEM((_EW,), jnp.int32),
            pltpu.VMEM((_EW,), jnp.int32),
            pltpu.VMEM((_EW,), jnp.float32),
            pltpu.VMEM((_EW,), jnp.float32),
            pltpu.VMEM((16,), jnp.float32),
            pltpu.SemaphoreType.DMA,
        ],
    )(_prep_body)
    return k(inv, px, py, pz, srcf, dstf)


def _seg_mean(d, s, n):
    tot = jax.ops.segment_sum(d, s, num_segments=n)
    cnt = jax.ops.segment_sum(jnp.ones((d.shape[0], 1), d.dtype), s, num_segments=n)
    return tot / jnp.maximum(cnt, 1.0)


def _seg_max0(d, s, n):
    m = jax.ops.segment_max(d, s, num_segments=n)
    return jnp.where(jnp.isfinite(m), m, 0.0)


def _dedup(key):
    TBL = 17 * (64 ** 3)
    tbl = jnp.full((TBL,), _N, jnp.int32).at[key].min(jnp.arange(_N, dtype=jnp.int32))
    return tbl[key]


def kernel(x, pos, edge_index, edge_attr, batch, params):
    valid = jnp.ones((_N,), jnp.int32)
    srcf = jnp.concatenate([edge_index[0], jnp.zeros((_EP - _E,), jnp.int32)]).reshape(_NW, _EW)
    dstf = jnp.concatenate([edge_index[1], jnp.full((_EP - _E,), _NP - 1, jnp.int32)]).reshape(_NW, _EW)
    eapad = jnp.concatenate([edge_attr, jnp.zeros((_EP - _E, 3), jnp.float32)], axis=0)
    ea0r = eapad[:, 0].reshape(_NW, _EW)
    ea1r = eapad[:, 1].reshape(_NW, _EW)
    ea2r = eapad[:, 2].reshape(_NW, _EW)
    for i, vs in enumerate(_VOXELS):
        g = int(np.ceil(_GRID / vs))
        c = jnp.clip(jnp.floor(pos / vs).astype(jnp.int32), 0, g - 1)
        key = batch * (g ** 3) + c[:, 0] * g * g + c[:, 1] * g + c[:, 2]
        inv = _dedup(key)
        p = params['conv%d' % (i + 1)]
        ci = p['Wn'].shape[0]
        co = p['Wn'].shape[1]
        xpad = jnp.zeros((_NP, ci), jnp.float32).at[:_N].set(x)
        xw = _pmatmul(xpad, p['Wn'])
        xr = _pmatmul(xpad, p['Wr'])[:_N]
        srcr = srcf.reshape(_NW, _CH * _NG, _KG)
        dstr = dstf.reshape(_NW, _CH * _NG, _KG)
        dst = dstf.reshape(-1)[:_E]
        F = 128
        xwp = xw if co >= F else jnp.pad(xw, ((0, 0), (0, F - co)))
        wep = p['We'] if co >= F else jnp.pad(p['We'], ((0, 0), (0, F - co)))
        cop = max(co, F)
        NCH = cop // F
        o = _sc_agg([xwp[:, k * F:(k + 1) * F] for k in range(NCH)],
                    srcr, dstr, ea0r, ea1r, ea2r, wep)
        osum = o[:, 0] + o[:, 1]
        tot = jnp.moveaxis(osum, 0, 1).reshape(_NP, cop)[:_N, :co]
        cnt = jax.ops.segment_sum(jnp.ones((_E, 1), jnp.float32), dst, num_segments=_N)
        agg = tot / jnp.maximum(cnt, 1.0)
        h = jax.nn.elu(agg + xr + p['b'])
        w = valid.astype(h.dtype)[:, None]
        cntv = jnp.maximum(jnp.sum(w), 1.0)
        mu = jnp.sum(h * w, axis=0) / cntv
        var = jnp.sum(((h - mu) ** 2) * w, axis=0) / cntv
        bp = params['bn%d' % (i + 1)]
        h = (h - mu) / jnp.sqrt(var + 1e-5) * bp['g'] + bp['b']
        x = _seg_max0(h, inv, _N)
        pos = _seg_mean(pos, inv, _N)
        batch = jnp.full((_N,), _B, jnp.int32).at[inv].set(batch)
        valid = jnp.zeros((_N,), jnp.int32).at[inv].max(valid)
        if i < 3:
            invp = jnp.concatenate([inv, jnp.arange(_N, _NP, dtype=jnp.int32)])
            pxp = jnp.concatenate([pos[:, 0], jnp.zeros((_NP - _N,), jnp.float32)])
            pyp = jnp.concatenate([pos[:, 1], jnp.zeros((_NP - _N,), jnp.float32)])
            pzp = jnp.concatenate([pos[:, 2], jnp.zeros((_NP - _N,), jnp.float32)])
            srcf, dstf, r0, r1, r2, pm = _sc_prep(invp, pxp, pyp, pzp, srcf, dstf)
            mx = jnp.max(pm) + 1e-9
            sc = 1.0 / (2.0 * mx)
            ea0r = r0 * sc + 0.5
            ea1r = r1 * sc + 0.5
            ea2r = r2 * sc + 0.5
    quad = (pos[:, 0] >= _GRID / 2).astype(jnp.int32) * 2 + (pos[:, 1] >= _GRID / 2).astype(jnp.int32)
    cl = batch * 4 + quad
    cl = jnp.where(valid > 0, cl, _B * 4)
    xp = _seg_max0(x, cl, _B * 4 + 1)[:_B * 4]
    xf = xp.reshape(_B, 512 * 4)
    h = xf @ params['lin1']['W'] + params['lin1']['b']
    return h @ params['lin2']['W'] + params['lin2']['b']


# speedup vs baseline: 1.1238x; 1.1238x over previous
import functools
import jax, jax.numpy as jnp
import numpy as np
from jax import lax
from jax.experimental import pallas as pl
from jax.experimental.pallas import tpu as pltpu
from jax.experimental.pallas import tpu_sc as plsc

_GRID = 256.0
_VOXELS = [4.0, 8.0, 16.0, 32.0]
_B = 16
_N = 10000
_NP = 10240
_E = 160000
_NC = 2
_NS = 16
_NW = _NC * _NS
_NT = _NP // _NW
_S = 2048
_K = 16


def _mm_body(x_ref, w_ref, o_ref):
    o_ref[...] = jnp.dot(x_ref[...], w_ref[...], preferred_element_type=jnp.float32)


def _pmatmul(x, w):
    M, Kd = x.shape
    co = w.shape[1]
    return pl.pallas_call(
        _mm_body,
        grid=(M // 1024,),
        in_specs=[pl.BlockSpec((1024, Kd), lambda i: (i, 0)),
                  pl.BlockSpec((Kd, co), lambda i: (0, 0))],
        out_specs=pl.BlockSpec((1024, co), lambda i: (i, 0)),
        out_shape=jax.ShapeDtypeStruct((M, co), jnp.float32),
    )(x, w)


_EP = 163840
_EW = _EP // _NW
_CH = 4
_EH = _EW // _CH
_KG = 32
_NG = _EH // _KG
_ZR = 8


def _agg_body(F, NCH, refs):
    xw_refs = refs[:NCH]
    (srcr_ref, dstr_ref, ea0_ref, ea1_ref, ea2_ref, we_ref, out_ref,
     sidx, didx, e0b, e1b, e2b, rows_a, rows_b, zbuf, wev, acc,
     sem_a, sem_b) = refs[NCH:]
    cid = lax.axis_index("c")
    sid = lax.axis_index("s")
    wid = sid * _NC + cid
    G = F // 16
    zero16 = jnp.zeros((16,), jnp.float32)
    pib = 'promise_in_bounds'
    ZS = _NP // _NS

    def zr(r, c):
        for g in range(G):
            zbuf[r, pl.ds(g * 16, 16)] = zero16
        return c
    lax.fori_loop(0, _ZR, zr, 0)

    for k in range(NCH):
        xw_ref = xw_refs[k]
        pltpu.sync_copy(we_ref.at[pl.ds(0, 3), pl.ds(k * F, F)], wev)
        def zc(t, c):
            pltpu.sync_copy(zbuf, acc.at[pl.ds(sid * ZS + t * _ZR, _ZR)])
            return c
        lax.fori_loop(0, ZS // _ZR, zc, 0)
        plsc.subcore_barrier()

        def chunk(h, c0_):
            pltpu.sync_copy(srcr_ref.at[wid, pl.ds(h * _NG, _NG)], sidx)
            pltpu.sync_copy(dstr_ref.at[wid, pl.ds(h * _NG, _NG)], didx)
            pltpu.sync_copy(ea0_ref.at[wid, pl.ds(h * _EH, _EH)], e0b)
            pltpu.sync_copy(ea1_ref.at[wid, pl.ds(h * _EH, _EH)], e1b)
            pltpu.sync_copy(ea2_ref.at[wid, pl.ds(h * _EH, _EH)], e2b)
            def proc(j, rows):
                wevs = [[wev[kk, pl.ds(g * 16, 16)] for g in range(G)] for kk in range(3)]
                for sub in range(_KG // 16):
                    e0v = e0b[pl.ds(j * _KG + sub * 16, 16)]
                    e1v = e1b[pl.ds(j * _KG + sub * 16, 16)]
                    e2v = e2b[pl.ds(j * _KG + sub * 16, 16)]

                    def edge(e16, c2):
                        lane = jnp.full((16,), e16, jnp.int32)
                        c0 = e0v.at[lane].get(mode=pib)
                        c1 = e1v.at[lane].get(mode=pib)
                        c2v = e2v.at[lane].get(mode=pib)
                        r = sub * 16 + e16
                        for g in range(G):
                            coef = c0 * wevs[0][g] + c1 * wevs[1][g] + c2v * wevs[2][g]
                            rows[r, pl.ds(g * 16, 16)] = rows[r, pl.ds(g * 16, 16)] * coef
                        return c2
                    lax.fori_loop(0, 16, edge, 0)
                pltpu.sync_copy(rows, acc.at[didx.at[j]], add=True)

            pltpu.async_copy(xw_ref.at[sidx.at[0]], rows_a, sem_a)

            def grp2(t, c):
                ja = 2 * t
                jb = 2 * t + 1
                pltpu.make_async_copy(xw_ref.at[sidx.at[ja]], rows_a, sem_a).wait()
                pltpu.async_copy(xw_ref.at[sidx.at[jb]], rows_b, sem_b)
                proc(ja, rows_a)
                pltpu.make_async_copy(xw_ref.at[sidx.at[jb]], rows_b, sem_b).wait()
                pltpu.async_copy(xw_ref.at[sidx.at[jnp.minimum(ja + 2, _NG - 1)]], rows_a, sem_a)
                proc(jb, rows_b)
                return c
            lax.fori_loop(0, _NG // 2, grp2, 0)
            pltpu.make_async_copy(xw_ref.at[sidx.at[_NG - 1]], rows_a, sem_a).wait()
            return c0_
        lax.fori_loop(0, _CH, chunk, 0)
        plsc.subcore_barrier()
        pltpu.sync_copy(acc.at[pl.ds(sid * ZS, ZS)], out_ref.at[k, cid, pl.ds(sid * ZS, ZS)])


def _sc_agg(xw_chunks, srcr, dstr, ea0r, ea1r, ea2r, we):
    F = xw_chunks[0].shape[1]
    NCH = len(xw_chunks)
    mesh = plsc.VectorSubcoreMesh(core_axis_name="c", subcore_axis_name="s")
    k = functools.partial(
        pl.kernel,
        mesh=mesh,
        out_type=jax.ShapeDtypeStruct((NCH, _NC, _NP, F), jnp.float32),
        scratch_types=[
            pltpu.VMEM((_NG, _KG), jnp.int32),
            pltpu.VMEM((_NG, _KG), jnp.int32),
            pltpu.VMEM((_EH,), jnp.float32),
            pltpu.VMEM((_EH,), jnp.float32),
            pltpu.VMEM((_EH,), jnp.float32),
            pltpu.VMEM((_KG, F), jnp.float32),
            pltpu.VMEM((_KG, F), jnp.float32),
            pltpu.VMEM((_ZR, F), jnp.float32),
            pltpu.VMEM((3, F), jnp.float32),
            pltpu.VMEM_SHARED((_NP, F), jnp.float32),
            pltpu.SemaphoreType.DMA,
            pltpu.SemaphoreType.DMA,
        ],
    )(lambda *refs: _agg_body(F, NCH, refs))
    return k(*xw_chunks, srcr, dstr, ea0r, ea1r, ea2r, we)


_GC = 128
_NGC = _EW // _GC


def _prep_body(inv_ref, px_ref, py_ref, pz_ref, srcf_ref, dstf_ref,
               so_ref, do_ref, r0_ref, r1_ref, r2_ref, pm_ref,
               sfl, dfl, snew, dnew, ps, pd, mxb, sem):
    cid = lax.axis_index("c")
    sid = lax.axis_index("s")
    wid = sid * _NC + cid

    pltpu.sync_copy(srcf_ref.at[wid], sfl)
    pltpu.sync_copy(dstf_ref.at[wid], dfl)

    def gat(table, idx, out):
        cps = []
        for t in range(_NGC):
            cps.append(pltpu.async_copy(table.at[idx.at[pl.ds(t * _GC, _GC)]],
                                        out.at[pl.ds(t * _GC, _GC)], sem))
        for c in cps:
            c.wait()

    gat(inv_ref, sfl, snew)
    gat(inv_ref, dfl, dnew)
    pltpu.sync_copy(snew, so_ref.at[wid])
    pltpu.sync_copy(dnew, do_ref.at[wid])

    maxv = jnp.zeros((16,), jnp.float32)
    for comp, (pref, rref) in enumerate([(px_ref, r0_ref), (py_ref, r1_ref), (pz_ref, r2_ref)]):
        gat(pref, snew, ps)
        gat(pref, dnew, pd)
        def rloop(t, mv):
            a = pd[pl.ds(t * 16, 16)] - ps[pl.ds(t * 16, 16)]
            ps[pl.ds(t * 16, 16)] = a
            return jnp.maximum(mv, jnp.abs(a))
        maxv = lax.fori_loop(0, _EW // 16, rloop, maxv)
        pltpu.sync_copy(ps, rref.at[wid])
    mxb[pl.ds(0, 16)] = maxv
    pltpu.sync_copy(mxb, pm_ref.at[wid])


def _sc_prep(inv, px, py, pz, srcf, dstf):
    mesh = plsc.VectorSubcoreMesh(core_axis_name="c", subcore_axis_name="s")
    k = functools.partial(
        pl.kernel,
        mesh=mesh,
        out_type=(jax.ShapeDtypeStruct((_NW, _EW), jnp.int32),
                  jax.ShapeDtypeStruct((_NW, _EW), jnp.int32),
                  jax.ShapeDtypeStruct((_NW, _EW), jnp.float32),
                  jax.ShapeDtypeStruct((_NW, _EW), jnp.float32),
                  jax.ShapeDtypeStruct((_NW, _EW), jnp.float32),
                  jax.ShapeDtypeStruct((_NW, 16), jnp.float32)),
        scratch_types=[
            pltpu.VMEM((_EW,), jnp.int32),
            pltpu.VMEM((_EW,), jnp.int32),
            pltpu.VMEM((_EW,), jnp.int32),
            pltpu.VMEM((_EW,), jnp.int32),
            pltpu.VMEM((_EW,), jnp.float32),
            pltpu.VMEM((_EW,), jnp.float32),
            pltpu.VMEM((16,), jnp.float32),
            pltpu.SemaphoreType.DMA,
        ],
    )(_prep_body)
    return k(inv, px, py, pz, srcf, dstf)


def _seg_mean(d, s, n):
    tot = jax.ops.segment_sum(d, s, num_segments=n)
    cnt = jax.ops.segment_sum(jnp.ones((d.shape[0], 1), d.dtype), s, num_segments=n)
    return tot / jnp.maximum(cnt, 1.0)


def _seg_max0(d, s, n):
    m = jax.ops.segment_max(d, s, num_segments=n)
    return jnp.where(jnp.isfinite(m), m, 0.0)


def _dedup(key):
    TBL = 17 * (64 ** 3)
    tbl = jnp.full((TBL,), _N, jnp.int32).at[key].min(jnp.arange(_N, dtype=jnp.int32))
    return tbl[key]


def kernel(x, pos, edge_index, edge_attr, batch, params):
    valid = jnp.ones((_N,), jnp.int32)
    srcf = jnp.concatenate([edge_index[0], jnp.zeros((_EP - _E,), jnp.int32)]).reshape(_NW, _EW)
    dstf = jnp.concatenate([edge_index[1], jnp.full((_EP - _E,), _NP - 1, jnp.int32)]).reshape(_NW, _EW)
    eapad = jnp.concatenate([edge_attr, jnp.zeros((_EP - _E, 3), jnp.float32)], axis=0)
    ea0r = eapad[:, 0].reshape(_NW, _EW)
    ea1r = eapad[:, 1].reshape(_NW, _EW)
    ea2r = eapad[:, 2].reshape(_NW, _EW)
    for i, vs in enumerate(_VOXELS):
        g = int(np.ceil(_GRID / vs))
        c = jnp.clip(jnp.floor(pos / vs).astype(jnp.int32), 0, g - 1)
        key = batch * (g ** 3) + c[:, 0] * g * g + c[:, 1] * g + c[:, 2]
        inv = _dedup(key)
        p = params['conv%d' % (i + 1)]
        ci = p['Wn'].shape[0]
        co = p['Wn'].shape[1]
        xpad = jnp.zeros((_NP, ci), jnp.float32).at[:_N].set(x)
        xw = _pmatmul(xpad, p['Wn'])
        xr = _pmatmul(xpad, p['Wr'])[:_N]
        srcr = srcf.reshape(_NW, _CH * _NG, _KG)
        dstr = dstf.reshape(_NW, _CH * _NG, _KG)
        dst = dstf.reshape(-1)[:_E]
        F = 128
        xwp = xw if co >= F else jnp.pad(xw, ((0, 0), (0, F - co)))
        wep = p['We'] if co >= F else jnp.pad(p['We'], ((0, 0), (0, F - co)))
        cop = max(co, F)
        NCH = cop // F
        o = _sc_agg([xwp[:, k * F:(k + 1) * F] for k in range(NCH)],
                    srcr, dstr, ea0r, ea1r, ea2r, wep)
        osum = o[:, 0] + o[:, 1]
        tot = jnp.moveaxis(osum, 0, 1).reshape(_NP, cop)[:_N, :co]
        cnt = jax.ops.segment_sum(jnp.ones((_E, 1), jnp.float32), dst, num_segments=_N)
        agg = tot / jnp.maximum(cnt, 1.0)
        h = jax.nn.elu(agg + xr + p['b'])
        w = valid.astype(h.dtype)[:, None]
        cntv = jnp.maximum(jnp.sum(w), 1.0)
        mu = jnp.sum(h * w, axis=0) / cntv
        var = jnp.sum(((h - mu) ** 2) * w, axis=0) / cntv
        bp = params['bn%d' % (i + 1)]
        h = (h - mu) / jnp.sqrt(var + 1e-5) * bp['g'] + bp['b']
        x = _seg_max0(h, inv, _N)
        pos = _seg_mean(pos, inv, _N)
        batch = jnp.full((_N,), _B, jnp.int32).at[inv].set(batch)
        valid = jnp.zeros((_N,), jnp.int32).at[inv].max(valid)
        if i < 3:
            invp = jnp.concatenate([inv, jnp.arange(_N, _NP, dtype=jnp.int32)])
            pxp = jnp.concatenate([pos[:, 0], jnp.zeros((_NP - _N,), jnp.float32)])
            pyp = jnp.concatenate([pos[:, 1], jnp.zeros((_NP - _N,), jnp.float32)])
            pzp = jnp.concatenate([pos[:, 2], jnp.zeros((_NP - _N,), jnp.float32)])
            srcf, dstf, r0, r1, r2, pm = _sc_prep(invp, pxp, pyp, pzp, srcf, dstf)
            mx = jnp.max(pm) + 1e-9
            sc = 1.0 / (2.0 * mx)
            ea0r = r0 * sc + 0.5
            ea1r = r1 * sc + 0.5
            ea2r = r2 * sc + 0.5
    quad = (pos[:, 0] >= _GRID / 2).astype(jnp.int32) * 2 + (pos[:, 1] >= _GRID / 2).astype(jnp.int32)
    cl = batch * 4 + quad
    cl = jnp.where(valid > 0, cl, _B * 4)
    xp = _seg_max0(x, cl, _B * 4 + 1)[:_B * 4]
    xf = xp.reshape(_B, 512 * 4)
    h = xf @ params['lin1']['W'] + params['lin1']['b']
    return h @ params['lin2']['W'] + params['lin2']['b']
